# baseline (speedup 1.0000x reference)
import functools

import jax
import jax.numpy as jnp
from jax import lax
from jax.experimental import pallas as pl
from jax.experimental.pallas import tpu as pltpu
from jax.experimental.pallas import tpu_sc as plsc

N = 10000
E = 320000
D = 128
NC = 2
NS = 16
NW = NC * NS

NPAD = 10240
CW = NPAD // NS
E_SCAN = E // NS
E_W = E // NW
G = 64

_Z16F = functools.partial(jnp.zeros, (16,), jnp.float32)
_Z16I = functools.partial(jnp.zeros, (16,), jnp.int32)


def _rsqrt_newton(x):
    xi = plsc.bitcast(x, jnp.int32)
    yi = jnp.int32(0x5F3759DF) - lax.shift_right_logical(xi, 1)
    y = plsc.bitcast(yi, jnp.float32)
    for _ in range(3):
        y = y * (1.5 - 0.5 * x * y * y)
    return y


def _norm_body(row_hbm, col_hbm, ew_hbm, dinv_out, norm_out,
               colb, ewb, priv, tmp, acc, disb, dinvb, disfull,
               row3, col3, ew3, norm3, degsh, dis_sh):
    c = lax.axis_index("c")
    s = lax.axis_index("s")

    pltpu.sync_copy(col_hbm.at[pl.ds(s * E_SCAN, E_SCAN)], colb)
    pltpu.sync_copy(ew_hbm.at[pl.ds(s * E_SCAN, E_SCAN)], ewb)

    def _zero(i, _):
        priv[pl.ds(i * 16, 16)] = _Z16F()
        return 0
    lax.fori_loop(0, NPAD // 16, _zero, 0)

    def _scat(i, _):
        cv = colb[pl.ds(i * 16, 16)]
        wv = ewb[pl.ds(i * 16, 16)]
        plsc.addupdate_scatter(priv, [cv], wv)
        return 0
    lax.fori_loop(0, E_SCAN // 16, _scat, 0)

    pltpu.sync_copy(priv, degsh.at[s])
    plsc.subcore_barrier()

    def _zacc(i, _):
        acc[pl.ds(i * 16, 16)] = _Z16F()
        return 0
    lax.fori_loop(0, CW // 16, _zacc, 0)
    for k in range(NS):
        pltpu.sync_copy(degsh.at[k, pl.ds(s * CW, CW)], tmp)

        def _add(i, _):
            acc[pl.ds(i * 16, 16)] = acc[pl.ds(i * 16, 16)] + tmp[pl.ds(i * 16, 16)]
            return 0
        lax.fori_loop(0, CW // 16, _add, 0)

    def _newton(i, _):
        x = acc[pl.ds(i * 16, 16)] + 1.0
        y = _rsqrt_newton(x)
        disb[pl.ds(i * 16, 16)] = y
        dinvb[pl.ds(i * 16, 16)] = y * y
        return 0
    lax.fori_loop(0, CW // 16, _newton, 0)

    pltpu.sync_copy(disb, dis_sh.at[pl.ds(s * CW, CW)])

    @pl.when(c == 0)
    def _():
        pltpu.sync_copy(dinvb, dinv_out.at[pl.ds(s * CW, CW)])

    plsc.subcore_barrier()
    pltpu.sync_copy(dis_sh, disfull)

    wid = c * NS + s
    base = wid * E_W
    pltpu.sync_copy(row_hbm.at[pl.ds(base, E_W)], row3)
    pltpu.sync_copy(col_hbm.at[pl.ds(base, E_W)], col3)
    pltpu.sync_copy(ew_hbm.at[pl.ds(base, E_W)], ew3)

    def _nrm(i, _):
        rv = row3[pl.ds(i * 16, 16)]
        cv = col3[pl.ds(i * 16, 16)]
        ev = ew3[pl.ds(i * 16, 16)]
        dr = plsc.load_gather(disfull, [rv])
        dc = plsc.load_gather(disfull, [cv])
        norm3[pl.ds(i * 16, 16)] = dr * ev * dc
        return 0
    lax.fori_loop(0, E_W // 16, _nrm, 0)
    pltpu.sync_copy(norm3, norm_out.at[pl.ds(base, E_W)])


NHALF = N // NC
SSC = 800
NSEG = E_SCAN // SSC
CCAP = SSC + G + 16
HROWS_A = 624
AROWS_A = 312


def _hop_body(h_hbm, row_hbm, col_hbm, norm_hbm, hout,
              hload, hpack, rows_s, cols_s, norm_s, rowc, colc, normc,
              rbufA, rbufB, sbuf, rowgA, rowgB, colg, zbuf,
              h_sh, acc, semA, semB):
    c = lax.axis_index("c")
    s = lax.axis_index("s")
    lo = c * NHALF

    def _stage(t, _):
        base = s * HROWS_A + t * 16
        pltpu.sync_copy(h_hbm.at[pl.ds(base, 16)], hload)

        def _pk(r2, _2):
            for half in range(2):
                for qq in range(4):
                    a = hload[2 * r2 + half, pl.ds(qq * 32, 16)]
                    b = hload[2 * r2 + half, pl.ds(qq * 32 + 16, 16)]
                    ab = plsc.pack(a, b, format=plsc.PackFormat.INTERLEAVED)
                    hpack[r2, pl.ds(half * 64 + qq * 16, 16)] = plsc.bitcast(
                        ab, jnp.int32)
            return 0
        lax.fori_loop(0, 8, _pk, 0)
        pltpu.sync_copy(hpack, h_sh.at[pl.ds(s * (HROWS_A // 2) + t * 8, 8)])
        return 0

    nhc = lax.select(s < 15, HROWS_A // 16, (N - 15 * HROWS_A) // 16)
    lax.fori_loop(0, nhc, _stage, 0)

    def _zz(i, _):
        for q in range(8):
            zbuf[i, pl.ds(q * 16, 16)] = _Z16F()
        return 0
    lax.fori_loop(0, 8, _zz, 0)

    def _za(t, _):
        pltpu.sync_copy(zbuf, acc.at[pl.ds(s * AROWS_A + t * 8, 8)])
        return 0
    nza = lax.select(s < 15, AROWS_A // 8, (NHALF - 15 * AROWS_A) // 8)
    lax.fori_loop(0, nza, _za, 0)

    def _zc(i, _):
        rowc[pl.ds(i * 16, 16)] = _Z16I()
        colc[pl.ds(i * 16, 16)] = _Z16I()
        normc[pl.ds(i * 16, 16)] = _Z16F()
        return 0
    lax.fori_loop(0, CCAP // 16, _zc, 0)
    plsc.subcore_barrier()

    lov = lax.broadcast(lo, (16,))
    hiv = lax.broadcast(lo + NHALF, (16,))

    def _gissue(rowg, rbuf, sem):
        pltpu.async_copy(h_sh.at[rowg.at[0]], rbuf, sem)

    def _gwait(rowg, rbuf, sem):
        pltpu.make_async_copy(h_sh.at[rowg.at[0]], rbuf, sem).wait()

    def _stepP(j, nchunks, cur_rowg, cur_rbuf, cur_sem,
               nxt_rowg, nxt_rbuf, nxt_sem):
        @pl.when(j + 1 < nchunks)
        def _():
            nb = (j + 1) * G
            for k in range(G // 16):
                nxt_rowg[0, pl.ds(k * 16, 16)] = lax.shift_right_logical(
                    rowc[pl.ds(nb + k * 16, 16)], 1)
            _gissue(nxt_rowg, nxt_rbuf, nxt_sem)

        _gwait(cur_rowg, cur_rbuf, cur_sem)

        def _scale(i, _):
            nv = normc[pl.ds(j * G + i * 16, 16)]
            pv = rowc[pl.ds(j * G + i * 16, 16)] & 1
            for e in range(16):
                sv = lax.broadcast(nv[e], (16,))
                pm = lax.broadcast(pv[e] == 1, (16,))
                r = i * 16 + e
                for qq in range(4):
                    xa = cur_rbuf[r, pl.ds(qq * 16, 16)]
                    xb = cur_rbuf[r, pl.ds(64 + qq * 16, 16)]
                    xi = jnp.where(pm, xb, xa)
                    x32 = plsc.bitcast(xi, jnp.bfloat16)
                    a, b = plsc.unpack(x32, format=plsc.PackFormat.INTERLEAVED)
                    sbuf[r, pl.ds(qq * 32, 16)] = a * sv
                    sbuf[r, pl.ds(qq * 32 + 16, 16)] = b * sv
            return 0
        lax.fori_loop(0, G // 16, _scale, 0)

        for k in range(G // 16):
            colg[0, pl.ds(k * 16, 16)] = colc[pl.ds(j * G + k * 16, 16)]
        pltpu.sync_copy(sbuf, acc.at[colg.at[0]], add=True)

    def _segment(seg, _):
        ebase = s * E_SCAN + seg * SSC
        pltpu.sync_copy(row_hbm.at[pl.ds(ebase, SSC)], rows_s)
        pltpu.sync_copy(col_hbm.at[pl.ds(ebase, SSC)], cols_s)
        pltpu.sync_copy(norm_hbm.at[pl.ds(ebase, SSC)], norm_s)

        def _zn(i, _2):
            normc[pl.ds(i * 16, 16)] = _Z16F()
            return 0
        lax.fori_loop(0, CCAP // 16, _zn, 0)

        def _compact(i, cnt):
            rv = rows_s[pl.ds(i * 16, 16)]
            cv = cols_s[pl.ds(i * 16, 16)]
            nv = norm_s[pl.ds(i * 16, 16)]
            mask = (cv >= lov) & (cv < hiv)
            plsc.store_compressed(rowc.at[pl.ds(cnt, 16)], rv, mask=mask)
            plsc.store_compressed(colc.at[pl.ds(cnt, 16)], cv - lov, mask=mask)
            plsc.store_compressed(normc.at[pl.ds(cnt, 16)], nv, mask=mask)
            return cnt + plsc.all_reduce_population_count(mask)[0]
        cnt = lax.fori_loop(0, SSC // 16, _compact, jnp.int32(0))

        nchunks = (cnt + (G - 1)) >> 6

        @pl.when(nchunks > 0)
        def _():
            for k in range(G // 16):
                rowgA[0, pl.ds(k * 16, 16)] = lax.shift_right_logical(
                    rowc[pl.ds(k * 16, 16)], 1)
            _gissue(rowgA, rbufA, semA)

        def _chunks(j, _2):
            @pl.when(j % 2 == 0)
            def _():
                _stepP(j, nchunks, rowgA, rbufA, semA, rowgB, rbufB, semB)

            @pl.when(j % 2 == 1)
            def _():
                _stepP(j, nchunks, rowgB, rbufB, semB, rowgA, rbufA, semA)
            return 0
        lax.fori_loop(0, nchunks, _chunks, 0)
        return 0
    lax.fori_loop(0, NSEG, _segment, 0)

    plsc.subcore_barrier()

    @pl.when(s < 15)
    def _():
        pltpu.sync_copy(acc.at[pl.ds(s * AROWS_A, AROWS_A)],
                        hout.at[pl.ds(lo + s * AROWS_A, AROWS_A)])

    @pl.when(s == 15)
    def _():
        pltpu.sync_copy(acc.at[pl.ds(15 * AROWS_A, NHALF - 15 * AROWS_A)],
                        hout.at[pl.ds(lo + 15 * AROWS_A, NHALF - 15 * AROWS_A)])


def _comb_body(p_ref, h_ref, dinv_ref, o_ref):
    o_ref[...] = p_ref[...] + dinv_ref[...] * h_ref[...]


def _mm_body(q_ref, h_ref, dinv_ref, w_ref, b_ref, o_ref):
    hh = q_ref[...] + dinv_ref[...] * h_ref[...]
    o_ref[...] = lax.dot_general(
        hh, w_ref[...], (((1,), (1,)), ((), ())),
        preferred_element_type=jnp.float32) + b_ref[...]


def _sc_mesh():
    return plsc.VectorSubcoreMesh(core_axis_name="c", subcore_axis_name="s")


def _make_norm():
    return pl.kernel(
        _norm_body,
        out_type=(jax.ShapeDtypeStruct((NPAD,), jnp.float32),
                  jax.ShapeDtypeStruct((E,), jnp.float32)),
        mesh=_sc_mesh(),
        compiler_params=pltpu.CompilerParams(needs_layout_passes=False),
        scratch_types=(
            pltpu.VMEM((E_SCAN,), jnp.int32),
            pltpu.VMEM((E_SCAN,), jnp.float32),
            pltpu.VMEM((NPAD,), jnp.float32),
            pltpu.VMEM((CW,), jnp.float32),
            pltpu.VMEM((CW,), jnp.float32),
            pltpu.VMEM((CW,), jnp.float32),
            pltpu.VMEM((CW,), jnp.float32),
            pltpu.VMEM((NPAD,), jnp.float32),
            pltpu.VMEM((E_W,), jnp.int32),
            pltpu.VMEM((E_W,), jnp.int32),
            pltpu.VMEM((E_W,), jnp.float32),
            pltpu.VMEM((E_W,), jnp.float32),
            pltpu.VMEM_SHARED((NS, NPAD), jnp.float32),
            pltpu.VMEM_SHARED((NPAD,), jnp.float32),
        ),
    )


def _make_hop():
    return pl.kernel(
        _hop_body,
        out_type=jax.ShapeDtypeStruct((N, D), jnp.float32),
        mesh=_sc_mesh(),
        compiler_params=pltpu.CompilerParams(needs_layout_passes=False),
        scratch_types=(
            pltpu.VMEM((16, D), jnp.float32),
            pltpu.VMEM((8, D), jnp.int32),
            pltpu.VMEM((SSC,), jnp.int32),
            pltpu.VMEM((SSC,), jnp.int32),
            pltpu.VMEM((SSC,), jnp.float32),
            pltpu.VMEM((CCAP,), jnp.int32),
            pltpu.VMEM((CCAP,), jnp.int32),
            pltpu.VMEM((CCAP,), jnp.float32),
            pltpu.VMEM((G, D), jnp.int32),
            pltpu.VMEM((G, D), jnp.int32),
            pltpu.VMEM((G, D), jnp.float32),
            pltpu.VMEM((1, G), jnp.int32),
            pltpu.VMEM((1, G), jnp.int32),
            pltpu.VMEM((1, G), jnp.int32),
            pltpu.VMEM((8, D), jnp.float32),
            pltpu.VMEM_SHARED((N // 2, D), jnp.int32),
            pltpu.VMEM_SHARED((NHALF, D), jnp.float32),
            pltpu.SemaphoreType.DMA,
            pltpu.SemaphoreType.DMA,
        ),
    )


def _comb(parts, h, dinv):
    return pl.pallas_call(
        _comb_body,
        out_shape=jax.ShapeDtypeStruct((N, D), jnp.float32),
        grid=(10,),
        in_specs=[
            pl.BlockSpec((N // 10, D), lambda i: (i, 0)),
            pl.BlockSpec((N // 10, D), lambda i: (i, 0)),
            pl.BlockSpec((N // 10, 1), lambda i: (i, 0)),
        ],
        out_specs=pl.BlockSpec((N // 10, D), lambda i: (i, 0)),
    )(parts, h, dinv)


def _mm(parts, h, dinv, Wt, b2):
    return pl.pallas_call(
        _mm_body,
        out_shape=jax.ShapeDtypeStruct((N, D), jnp.float32),
        grid=(10,),
        in_specs=[
            pl.BlockSpec((N // 10, D), lambda i: (i, 0)),
            pl.BlockSpec((N // 10, D), lambda i: (i, 0)),
            pl.BlockSpec((N // 10, 1), lambda i: (i, 0)),
            pl.BlockSpec((D, D), lambda i: (0, 0)),
            pl.BlockSpec((1, D), lambda i: (0, 0)),
        ],
        out_specs=pl.BlockSpec((N // 10, D), lambda i: (i, 0)),
    )(parts, h, dinv, Wt, b2)


def kernel(x, edge_index, edge_weight, W, b):
    row = edge_index[0]
    col = edge_index[1]
    dinv_pad, normv = _make_norm()(row, col, edge_weight)
    dinv = dinv_pad[:N].reshape(N, 1)
    hop = _make_hop()
    p1 = hop(x, row, col, normv)
    h1 = _comb(p1, x, dinv)
    p2 = hop(h1, row, col, normv)
    return _mm(p2, h1, dinv, W, b.reshape(1, D))
